# baseline (speedup 1.0000x reference)
import jax
import jax.numpy as jnp
from jax import lax
from jax.experimental import pallas as pl
from jax.experimental.pallas import tpu as pltpu
from jax.experimental.pallas import tpu_sc as plsc

BATCH = 16384
HIST = 50
D = 32
NC = 2
NS = 16
NW = NC * NS
ROWS_PW = BATCH // NW
JB = ROWS_PW // 128
NPAIR = HIST // 2


def _transpose_block(g_ref, t_ref, rb, dcols):
    for d in range(D):
        for k in range(8):
            v = plsc.load_gather(g_ref, [rb[k], dcols[d]])
            t_ref[d >> 3, d & 7, pl.ds(k * 16, 16)] = v


def _gather_body(table_hbm, idx_hbm, out_hbm, idx_v, idxt_v, g0, g1, t_v,
                 s0, s1):
    wid = lax.axis_index("s") * NC + lax.axis_index("c")
    base = wid * ROWS_PW
    pltpu.sync_copy(idx_hbm.at[pl.ds(base, ROWS_PW)], idx_v)

    iota = lax.iota(jnp.int32, 16)
    rb = [iota + (16 * k) for k in range(32)]
    dcols = [jnp.full((16,), d, jnp.int32) for d in range(D)]

    def hloop(h, carry):
        hcol = jnp.full((16,), h, jnp.int32)
        for k in range(32):
            v = plsc.load_gather(idx_v, [rb[k], hcol])
            idxt_v[h, pl.ds(k * 16, 16)] = v
        return carry

    lax.fori_loop(0, HIST, hloop, 0)

    def fire(h, jj, gbuf, sem):
        pltpu.async_copy(
            table_hbm.at[idxt_v.at[h, pl.ds(128 * jj, 128)]], gbuf, sem
        )

    def drain(gbuf, sem):
        pltpu.make_async_copy(
            table_hbm.at[idxt_v.at[0, pl.ds(0, 128)]], gbuf, sem
        ).wait()

    for jj in range(JB):
        j = wid * JB + jj
        fire(0, jj, g0, s0)
        fire(1, jj, g1, s1)

        def pair(t, carry):
            h0 = 2 * t
            drain(g0, s0)
            _transpose_block(g0, t_v, rb, dcols)
            pltpu.sync_copy(t_v, out_hbm.at[h0, :, j])

            @pl.when(h0 + 2 < HIST)
            def _():
                fire(h0 + 2, jj, g0, s0)

            drain(g1, s1)
            _transpose_block(g1, t_v, rb, dcols)
            pltpu.sync_copy(t_v, out_hbm.at[h0 + 1, :, j])

            @pl.when(h0 + 3 < HIST)
            def _():
                fire(h0 + 3, jj, g1, s1)

            return carry

        lax.fori_loop(0, NPAIR, pair, 0)


def kernel(input_ids, weight_shard):
    idx = input_ids.astype(jnp.int32)
    mesh = plsc.VectorSubcoreMesh(core_axis_name="c", subcore_axis_name="s")
    out5 = pl.kernel(
        _gather_body,
        out_type=jax.ShapeDtypeStruct((HIST, D // 8, BATCH // 128, 8, 128),
                                      jnp.float32),
        mesh=mesh,
        scratch_types=[
            pltpu.VMEM((ROWS_PW, HIST), jnp.int32),
            pltpu.VMEM((HIST, ROWS_PW), jnp.int32),
            pltpu.VMEM((128, D), jnp.float32),
            pltpu.VMEM((128, D), jnp.float32),
            pltpu.VMEM((D // 8, 8, 128), jnp.float32),
            pltpu.SemaphoreType.DMA,
            pltpu.SemaphoreType.DMA,
        ],
        compiler_params=pltpu.CompilerParams(
            use_tc_tiling_on_sc=False, needs_layout_passes=False
        ),
    )(weight_shard, idx)
    return jnp.transpose(out5, (2, 4, 0, 1, 3)).reshape(BATCH, HIST, D)


# speedup vs baseline: 1.5776x; 1.5776x over previous
import jax
import jax.numpy as jnp
from jax import lax
from jax.experimental import pallas as pl
from jax.experimental.pallas import tpu as pltpu
from jax.experimental.pallas import tpu_sc as plsc

BATCH = 16384
HIST = 50
D = 32
NC = 2
NS = 16
NW = NC * NS
ROWS_PW = BATCH // NW
JB = ROWS_PW // 128
NPAIR = HIST // 2


def _transpose_block(g_ref, t_ref, dv_lo, dv_hi):
    for c in range(128):
        cv = jnp.full((16,), c, jnp.int32)
        v_lo = g_ref[c, pl.ds(0, 16)]
        v_hi = g_ref[c, pl.ds(16, 16)]
        plsc.store_scatter(t_ref, [dv_lo, cv], v_lo)
        plsc.store_scatter(t_ref, [dv_hi, cv], v_hi)


def _gather_body(table_hbm, idx_hbm, out_hbm, idx_v, idxt_v, g0, g1, t_v,
                 s0, s1):
    wid = lax.axis_index("s") * NC + lax.axis_index("c")
    base = wid * ROWS_PW
    pltpu.sync_copy(idx_hbm.at[pl.ds(base, ROWS_PW)], idx_v)

    iota = lax.iota(jnp.int32, 16)
    rb = [iota + (16 * k) for k in range(32)]
    dv_lo = iota
    dv_hi = iota + 16

    def hloop(h, carry):
        hcol = jnp.full((16,), h, jnp.int32)
        for k in range(32):
            v = plsc.load_gather(idx_v, [rb[k], hcol])
            idxt_v[h, pl.ds(k * 16, 16)] = v
        return carry

    lax.fori_loop(0, HIST, hloop, 0)

    def fire(h, jj, gbuf, sem):
        pltpu.async_copy(
            table_hbm.at[idxt_v.at[h, pl.ds(128 * jj, 128)]], gbuf, sem
        )

    def drain(gbuf, sem):
        pltpu.make_async_copy(
            table_hbm.at[idxt_v.at[0, pl.ds(0, 128)]], gbuf, sem
        ).wait()

    for jj in range(JB):
        j = wid * JB + jj
        fire(0, jj, g0, s0)
        fire(1, jj, g1, s1)

        def pair(t, carry):
            h0 = 2 * t
            drain(g0, s0)
            _transpose_block(g0, t_v, dv_lo, dv_hi)
            for d0 in range(4):
                pltpu.sync_copy(t_v.at[pl.ds(8 * d0, 8), pl.ds(0, 128)],
                                out_hbm.at[h0, d0, j])

            @pl.when(h0 + 2 < HIST)
            def _():
                fire(h0 + 2, jj, g0, s0)

            drain(g1, s1)
            _transpose_block(g1, t_v, dv_lo, dv_hi)
            for d0 in range(4):
                pltpu.sync_copy(t_v.at[pl.ds(8 * d0, 8), pl.ds(0, 128)],
                                out_hbm.at[h0 + 1, d0, j])

            @pl.when(h0 + 3 < HIST)
            def _():
                fire(h0 + 3, jj, g1, s1)

            return carry

        lax.fori_loop(0, NPAIR, pair, 0)


def kernel(input_ids, weight_shard):
    idx = input_ids.astype(jnp.int32)
    mesh = plsc.VectorSubcoreMesh(core_axis_name="c", subcore_axis_name="s")
    out5 = pl.kernel(
        _gather_body,
        out_type=jax.ShapeDtypeStruct((HIST, D // 8, BATCH // 128, 8, 128),
                                      jnp.float32),
        mesh=mesh,
        scratch_types=[
            pltpu.VMEM((ROWS_PW, HIST), jnp.int32),
            pltpu.VMEM((HIST, ROWS_PW), jnp.int32),
            pltpu.VMEM((128, D), jnp.float32),
            pltpu.VMEM((128, D), jnp.float32),
            pltpu.VMEM((D, 129), jnp.float32),
            pltpu.SemaphoreType.DMA,
            pltpu.SemaphoreType.DMA,
        ],
        compiler_params=pltpu.CompilerParams(
            use_tc_tiling_on_sc=False, needs_layout_passes=False
        ),
    )(weight_shard, idx)
    return jnp.transpose(out5, (2, 4, 0, 1, 3)).reshape(BATCH, HIST, D)


# speedup vs baseline: 1.6977x; 1.0761x over previous
import jax
import jax.numpy as jnp
from jax import lax
from jax.experimental import pallas as pl
from jax.experimental.pallas import tpu as pltpu
from jax.experimental.pallas import tpu_sc as plsc

BATCH = 16384
HIST = 50
D = 32
NC = 2
NS = 16
NW = NC * NS
ROWS_PW = BATCH // NW
JB = ROWS_PW // 128
NPAIR = HIST // 2


def _transpose_block(g_ref, t_ref, dv_lo, dv_hi):
    for c in range(128):
        cv = jnp.full((16,), c, jnp.int32)
        v_lo = g_ref[c, pl.ds(0, 16)]
        v_hi = g_ref[c, pl.ds(16, 16)]
        plsc.store_scatter(t_ref, [dv_lo, cv], v_lo)
        plsc.store_scatter(t_ref, [dv_hi, cv], v_hi)


def _gather_body(table_hbm, idx_hbm, out_hbm, idx_v, idxt_v, g0, g1, t0, t1,
                 s0, s1, w0, w1):
    wid = lax.axis_index("s") * NC + lax.axis_index("c")
    base = wid * ROWS_PW
    pltpu.sync_copy(idx_hbm.at[pl.ds(base, ROWS_PW)], idx_v)

    iota = lax.iota(jnp.int32, 16)
    rb = [iota + (16 * k) for k in range(32)]
    dv_lo = iota
    dv_hi = iota + 16

    def hloop(h, carry):
        hcol = jnp.full((16,), h, jnp.int32)
        for k in range(32):
            v = plsc.load_gather(idx_v, [rb[k], hcol])
            idxt_v[h, pl.ds(k * 16, 16)] = v
        return carry

    lax.fori_loop(0, HIST, hloop, 0)

    def fire(h, jj, gbuf, sem):
        pltpu.async_copy(
            table_hbm.at[idxt_v.at[h, pl.ds(128 * jj, 128)]], gbuf, sem
        )

    def drain(gbuf, sem):
        pltpu.make_async_copy(
            table_hbm.at[idxt_v.at[0, pl.ds(0, 128)]], gbuf, sem
        ).wait()

    for jj in range(JB):
        j = wid * JB + jj
        fire(0, jj, g0, s0)
        fire(1, jj, g1, s1)

        def wdrain(tbuf, sem):
            for d0 in range(4):
                pltpu.make_async_copy(
                    tbuf.at[pl.ds(8 * d0, 8), pl.ds(0, 128)],
                    out_hbm.at[0, d0, 0], sem
                ).wait()

        def pair(t, carry):
            h0 = 2 * t
            drain(g0, s0)

            @pl.when(t > 0)
            def _():
                wdrain(t0, w0)

            _transpose_block(g0, t0, dv_lo, dv_hi)

            @pl.when(h0 + 2 < HIST)
            def _():
                fire(h0 + 2, jj, g0, s0)

            for d0 in range(4):
                pltpu.async_copy(t0.at[pl.ds(8 * d0, 8), pl.ds(0, 128)],
                                 out_hbm.at[h0, d0, j], w0)

            drain(g1, s1)

            @pl.when(t > 0)
            def _():
                wdrain(t1, w1)

            _transpose_block(g1, t1, dv_lo, dv_hi)

            @pl.when(h0 + 3 < HIST)
            def _():
                fire(h0 + 3, jj, g1, s1)

            for d0 in range(4):
                pltpu.async_copy(t1.at[pl.ds(8 * d0, 8), pl.ds(0, 128)],
                                 out_hbm.at[h0 + 1, d0, j], w1)

            return carry

        lax.fori_loop(0, NPAIR, pair, 0)
        wdrain(t0, w0)
        wdrain(t1, w1)


def kernel(input_ids, weight_shard):
    idx = input_ids.astype(jnp.int32)
    mesh = plsc.VectorSubcoreMesh(core_axis_name="c", subcore_axis_name="s")
    out5 = pl.kernel(
        _gather_body,
        out_type=jax.ShapeDtypeStruct((HIST, D // 8, BATCH // 128, 8, 128),
                                      jnp.float32),
        mesh=mesh,
        scratch_types=[
            pltpu.VMEM((ROWS_PW, HIST), jnp.int32),
            pltpu.VMEM((HIST, ROWS_PW), jnp.int32),
            pltpu.VMEM((128, D), jnp.float32),
            pltpu.VMEM((128, D), jnp.float32),
            pltpu.VMEM((D, 129), jnp.float32),
            pltpu.VMEM((D, 129), jnp.float32),
            pltpu.SemaphoreType.DMA,
            pltpu.SemaphoreType.DMA,
            pltpu.SemaphoreType.DMA,
            pltpu.SemaphoreType.DMA,
        ],
        compiler_params=pltpu.CompilerParams(
            use_tc_tiling_on_sc=False, needs_layout_passes=False
        ),
    )(weight_shard, idx)
    return jnp.transpose(out5, (2, 4, 0, 1, 3)).reshape(BATCH, HIST, D)


# speedup vs baseline: 1.7006x; 1.0017x over previous
import jax
import jax.numpy as jnp
from jax import lax
from jax.experimental import pallas as pl
from jax.experimental.pallas import tpu as pltpu
from jax.experimental.pallas import tpu_sc as plsc

BATCH = 16384
HIST = 50
D = 32
NC = 2
NS = 16
NW = NC * NS
ROWS_PW = BATCH // NW
JB = ROWS_PW // 128
NPAIR = HIST // 2


def _transpose_block(g_ref, t_ref, dv_lo, dv_hi):
    for c in range(128):
        cv = jnp.full((16,), c, jnp.int32)
        v_lo = g_ref[c, pl.ds(0, 16)]
        v_hi = g_ref[c, pl.ds(16, 16)]
        plsc.store_scatter(t_ref, [dv_lo, cv], v_lo)
        plsc.store_scatter(t_ref, [dv_hi, cv], v_hi)


def _gather_body(table_hbm, idx_hbm, out_hbm, idx_v, idxt_v, g0, g1, t0, t1,
                 s0, s1, w0, w1):
    wid = lax.axis_index("s") * NC + lax.axis_index("c")
    base = wid * ROWS_PW
    pltpu.sync_copy(idx_hbm.at[pl.ds(base, ROWS_PW)], idx_v)

    iota = lax.iota(jnp.int32, 16)
    rb = [iota + (16 * k) for k in range(32)]
    dv_lo = iota
    dv_hi = iota + 16

    def hloop(h, carry):
        hcol = jnp.full((16,), h, jnp.int32)
        for k in range(32):
            v = plsc.load_gather(idx_v, [rb[k], hcol])
            idxt_v[h, pl.ds(k * 16, 16)] = v
        return carry

    def fire(h, jj, gbuf, sem):
        pltpu.async_copy(
            table_hbm.at[idxt_v.at[h, pl.ds(128 * jj, 128)]], gbuf, sem
        )

    def drain(gbuf, sem):
        pltpu.make_async_copy(
            table_hbm.at[idxt_v.at[0, pl.ds(0, 128)]], gbuf, sem
        ).wait()

    hloop(0, 0)
    hloop(1, 0)
    fire(0, 0, g0, s0)
    fire(1, 0, g1, s1)
    lax.fori_loop(2, HIST, hloop, 0)

    for jj in range(JB):
        j = wid * JB + jj
        if jj > 0:
            fire(0, jj, g0, s0)
            fire(1, jj, g1, s1)

        def wdrain(tbuf, sem):
            for d0 in range(4):
                pltpu.make_async_copy(
                    tbuf.at[pl.ds(8 * d0, 8), pl.ds(0, 128)],
                    out_hbm.at[0, d0, 0], sem
                ).wait()

        def pair(t, carry):
            h0 = 2 * t
            drain(g0, s0)

            @pl.when(t > 0)
            def _():
                wdrain(t0, w0)

            _transpose_block(g0, t0, dv_lo, dv_hi)

            @pl.when(h0 + 2 < HIST)
            def _():
                fire(h0 + 2, jj, g0, s0)

            for d0 in range(4):
                pltpu.async_copy(t0.at[pl.ds(8 * d0, 8), pl.ds(0, 128)],
                                 out_hbm.at[h0, d0, j], w0)

            drain(g1, s1)

            @pl.when(t > 0)
            def _():
                wdrain(t1, w1)

            _transpose_block(g1, t1, dv_lo, dv_hi)

            @pl.when(h0 + 3 < HIST)
            def _():
                fire(h0 + 3, jj, g1, s1)

            for d0 in range(4):
                pltpu.async_copy(t1.at[pl.ds(8 * d0, 8), pl.ds(0, 128)],
                                 out_hbm.at[h0 + 1, d0, j], w1)

            return carry

        lax.fori_loop(0, NPAIR, pair, 0)
        wdrain(t0, w0)
        wdrain(t1, w1)


def kernel(input_ids, weight_shard):
    idx = input_ids.astype(jnp.int32)
    mesh = plsc.VectorSubcoreMesh(core_axis_name="c", subcore_axis_name="s")
    out5 = pl.kernel(
        _gather_body,
        out_type=jax.ShapeDtypeStruct((HIST, D // 8, BATCH // 128, 8, 128),
                                      jnp.float32),
        mesh=mesh,
        scratch_types=[
            pltpu.VMEM((ROWS_PW, HIST), jnp.int32),
            pltpu.VMEM((HIST, ROWS_PW), jnp.int32),
            pltpu.VMEM((128, D), jnp.float32),
            pltpu.VMEM((128, D), jnp.float32),
            pltpu.VMEM((D, 129), jnp.float32),
            pltpu.VMEM((D, 129), jnp.float32),
            pltpu.SemaphoreType.DMA,
            pltpu.SemaphoreType.DMA,
            pltpu.SemaphoreType.DMA,
            pltpu.SemaphoreType.DMA,
        ],
        compiler_params=pltpu.CompilerParams(
            use_tc_tiling_on_sc=False, needs_layout_passes=False
        ),
    )(weight_shard, idx)
    return jnp.transpose(out5, (2, 4, 0, 1, 3)).reshape(BATCH, HIST, D)
